# Initial kernel scaffold; baseline (speedup 1.0000x reference)
import jax
import jax.numpy as jnp
from jax.experimental import pallas as pl


def kernel(domain_mapping, x):
    raise NotImplementedError("write your pallas kernel here")


# speedup vs baseline: 1.2920x; 1.2920x over previous
import functools

import jax
import jax.numpy as jnp
from jax import lax
from jax.experimental import pallas as pl
from jax.experimental.pallas import tpu as pltpu
from jax.experimental.pallas import tpu_sc as plsc

_NC = 2
_NS = 16
_NW = _NC * _NS
_W = 128


@functools.partial(jax.jit, static_argnames=())
def _sc_gather(table, xr):
    rows, w = xr.shape
    per_w = rows // _NW
    mesh = plsc.VectorSubcoreMesh(core_axis_name="c", subcore_axis_name="s")

    @functools.partial(
        pl.kernel,
        mesh=mesh,
        out_type=jax.ShapeDtypeStruct((rows, w), jnp.int32),
        scratch_types=[
            pltpu.VMEM((per_w, w), jnp.int32),
            pltpu.VMEM((per_w, w), jnp.int32),
            pltpu.SemaphoreType.DMA,
        ],
    )
    def body(table_hbm, xr_hbm, out_hbm, idx_v, out_v, sem):
        wid = lax.axis_index("s") * _NC + lax.axis_index("c")
        base = wid * per_w
        pltpu.sync_copy(xr_hbm.at[pl.ds(base, per_w)], idx_v)

        @pl.loop(0, per_w)
        def _fire(j):
            pltpu.async_copy(table_hbm.at[idx_v.at[j]], out_v.at[j], sem)

        pltpu.make_async_copy(out_hbm.at[pl.ds(base, per_w)], out_v, sem).wait()
        pltpu.sync_copy(out_v, out_hbm.at[pl.ds(base, per_w)])

    return body(table, xr)


def kernel(domain_mapping, x):
    b, f = x.shape
    xr = x.reshape((b * f) // _W, _W)
    out = _sc_gather(domain_mapping, xr)
    return out.reshape(b, f)
